# baseline (speedup 1.0000x reference)
import functools

import jax
import jax.numpy as jnp
from jax import lax
from jax.experimental import pallas as pl
from jax.experimental.pallas import tpu as pltpu
from jax.experimental.pallas import tpu_sc as plsc

_CHUNK = 128


@functools.cache
def _make_gather(V, D, B):
    info = plsc.get_sparse_core_info()
    NC, NS = info.num_cores, info.num_subcores
    NW = NC * NS
    assert B % (NW * _CHUNK) == 0
    b_per_w = B // NW
    n_chunks = b_per_w // _CHUNK
    mesh = plsc.VectorSubcoreMesh(core_axis_name="c", subcore_axis_name="s")

    @functools.partial(
        pl.kernel,
        mesh=mesh,
        out_type=jax.ShapeDtypeStruct((B, D), jnp.float32),
        compiler_params=pltpu.CompilerParams(use_tc_tiling_on_sc=False),
        scratch_types=[
            pltpu.VMEM((n_chunks, _CHUNK), jnp.int32),
            pltpu.VMEM((b_per_w, D), jnp.float32),
            pltpu.SemaphoreType.DMA,
        ],
    )
    def gather_kernel(table_hbm, idx_hbm, out_hbm, idx_v, rows_v, sem):
        wid = lax.axis_index("s") * NC + lax.axis_index("c")
        pltpu.sync_copy(idx_hbm.at[pl.ds(wid * n_chunks, n_chunks)], idx_v)
        copies = [
            pltpu.async_copy(
                table_hbm.at[idx_v.at[j]],
                rows_v.at[pl.ds(j * _CHUNK, _CHUNK)],
                sem,
            )
            for j in range(n_chunks)
        ]
        for c in copies:
            c.wait()
        pltpu.sync_copy(rows_v, out_hbm.at[pl.ds(wid * b_per_w, b_per_w)])

    return gather_kernel


def kernel(dist, idx):
    V, D = dist.shape
    B = idx.shape[0]
    idx2 = idx.reshape(B // _CHUNK, _CHUNK)
    return _make_gather(V, D, B)(dist, idx2)


# speedup vs baseline: 6.3203x; 6.3203x over previous
import functools

import jax
import jax.numpy as jnp
from jax import lax
from jax.experimental import pallas as pl
from jax.experimental.pallas import tpu as pltpu
from jax.experimental.pallas import tpu_sc as plsc

_L = 16
_CH = 16
_TW = 128


@functools.cache
def _make_gather(V, D, B):
    info = plsc.get_sparse_core_info()
    NC, NS = info.num_cores, info.num_subcores
    NW = NC * NS
    b_per_w = B // NW
    n_chunks = b_per_w // _CH
    mesh = plsc.VectorSubcoreMesh(core_axis_name="c", subcore_axis_name="s")

    @functools.partial(
        pl.kernel,
        mesh=mesh,
        compiler_params=pltpu.CompilerParams(needs_layout_passes=False),
        out_type=jax.ShapeDtypeStruct((D, B), jnp.float32),
        scratch_types=[
            pltpu.VMEM((b_per_w,), jnp.int32),
            pltpu.VMEM((2, _CH, D, _TW), jnp.float32),
            pltpu.VMEM((D, b_per_w), jnp.float32),
            pltpu.SemaphoreType.DMA,
        ],
    )
    def gather_kernel(table_t, idx_hbm, out_t, idx_v, slab_v, out_v, sem):
        wid = lax.axis_index("s") * NC + lax.axis_index("c")
        base_b = pl.multiple_of(wid * b_per_w, b_per_w)
        pltpu.sync_copy(idx_hbm.at[pl.ds(base_b, b_per_w)], idx_v)

        iota = lax.iota(jnp.int32, _L)

        def issue(g):
            half = g % 2
            tc_vec = lax.shift_right_logical(
                idx_v[pl.ds(g * _CH, _CH)], jnp.int32(7)
            )
            copies = []
            for t in range(_CH):
                lane0 = pl.multiple_of(tc_vec[t] * _TW, _TW)
                copies.append(
                    pltpu.async_copy(
                        table_t.at[:, pl.ds(lane0, _TW)],
                        slab_v.at[half, t],
                        sem,
                    )
                )
            return copies

        def extract(g):
            half = g % 2
            lane_vec = jnp.bitwise_and(
                idx_v[pl.ds(g * _CH, _CH)], jnp.int32(_TW - 1)
            )
            for j in range(D):
                jvec = jnp.full((_L,), j, jnp.int32)
                vals = plsc.load_gather(slab_v.at[half], [iota, jvec, lane_vec])
                out_v[j, pl.ds(g * _CH, _L)] = vals

        pending = issue(0)
        for g in range(1, n_chunks):
            nxt = issue(g)
            for c in pending:
                c.wait()
            extract(g - 1)
            pending = nxt
        for c in pending:
            c.wait()
        extract(n_chunks - 1)

        pltpu.sync_copy(out_v, out_t.at[:, pl.ds(base_b, b_per_w)])

    return gather_kernel


def kernel(dist, idx):
    V, D = dist.shape
    B = idx.shape[0]
    out_t = _make_gather(V, D, B)(dist.T, idx)
    return out_t.T
